# baseline (speedup 1.0000x reference)
import functools
import jax
import jax.numpy as jnp
from jax import lax
from jax.experimental import pallas as pl
from jax.experimental.pallas import tpu as pltpu
from jax.experimental.pallas import tpu_sc as plsc

NBUF = 4


def kernel(encoded_patches, pos_table):
    B, P, D = encoded_patches.shape
    info = plsc.get_sparse_core_info()
    NC, NS, L = info.num_cores, info.num_subcores, info.num_lanes
    NW = NC * NS
    RP = P // NW
    NVEC = D // L
    CH = RP * D

    mesh = plsc.VectorSubcoreMesh(core_axis_name="c", subcore_axis_name="s")

    @functools.partial(
        pl.kernel,
        mesh=mesh,
        out_type=jax.ShapeDtypeStruct((B * P * D,), jnp.float32),
        scratch_types=(
            [pltpu.VMEM((CH,), jnp.float32)]
            + [pltpu.VMEM((CH,), jnp.float32)] * NBUF
            + [pltpu.SemaphoreType.DMA] * NBUF
            + [pltpu.SemaphoreType.DMA] * NBUF
        ),
    )
    def sc_add(x_hbm, t_hbm, o_hbm, tbuf, *rest):
        bufs = rest[:NBUF]
        in_sems = rest[NBUF:2 * NBUF]
        out_sems = rest[2 * NBUF:3 * NBUF]

        w = lax.axis_index("s") * NC + lax.axis_index("c")
        t0 = w * CH
        pltpu.sync_copy(t_hbm.at[pl.ds(t0, CH)], tbuf)

        def add_rows(buf):
            def row(i, _):
                for j in range(NVEC):
                    sl = pl.ds(i * D + j * L, L)
                    plsc.addupdate(buf.at[sl], tbuf[sl])
                return 0
            lax.fori_loop(0, RP, row, 0)

        def outer(g, _):
            base = g * NBUF
            for b in range(NBUF):
                @pl.when(g > 0)
                def _():
                    pltpu.make_async_copy(
                        bufs[b],
                        o_hbm.at[pl.ds((base - NBUF + b) * P * D + t0, CH)],
                        out_sems[b],
                    ).wait()
                pltpu.async_copy(
                    x_hbm.at[pl.ds((base + b) * P * D + t0, CH)], bufs[b], in_sems[b]
                )
            for b in range(NBUF):
                pltpu.make_async_copy(
                    x_hbm.at[pl.ds((base + b) * P * D + t0, CH)], bufs[b], in_sems[b]
                ).wait()
                add_rows(bufs[b])
                pltpu.async_copy(
                    bufs[b], o_hbm.at[pl.ds((base + b) * P * D + t0, CH)], out_sems[b]
                )
            return 0

        lax.fori_loop(0, B // NBUF, outer, 0)
        for b in range(NBUF):
            pltpu.make_async_copy(
                bufs[b],
                o_hbm.at[pl.ds((B - NBUF + b) * P * D + t0, CH)],
                out_sems[b],
            ).wait()

    out = sc_add(encoded_patches.reshape(-1), pos_table.reshape(-1))
    return out.reshape(B, P, D)


# speedup vs baseline: 1.0030x; 1.0030x over previous
import functools
import jax
import jax.numpy as jnp
from jax import lax
from jax.experimental import pallas as pl
from jax.experimental.pallas import tpu as pltpu
from jax.experimental.pallas import tpu_sc as plsc

NBUF = 4


def kernel(encoded_patches, pos_table):
    B, P, D = encoded_patches.shape
    info = plsc.get_sparse_core_info()
    NC, NS, L = info.num_cores, info.num_subcores, info.num_lanes
    NW = NC * NS
    RP = P // NW
    NVEC = D // L
    CH = RP * D

    mesh = plsc.VectorSubcoreMesh(core_axis_name="c", subcore_axis_name="s")

    @functools.partial(
        pl.kernel,
        mesh=mesh,
        out_type=jax.ShapeDtypeStruct((B * P * D,), jnp.float32),
        scratch_types=(
            [pltpu.VMEM((CH,), jnp.float32)]
            + [pltpu.VMEM((CH,), jnp.float32)] * NBUF
            + [pltpu.SemaphoreType.DMA] * NBUF
            + [pltpu.SemaphoreType.DMA] * NBUF
        ),
    )
    def sc_add(x_hbm, t_hbm, o_hbm, tbuf, *rest):
        bufs = rest[:NBUF]
        in_sems = rest[NBUF:2 * NBUF]
        out_sems = rest[2 * NBUF:3 * NBUF]

        w = lax.axis_index("s") * NC + lax.axis_index("c")
        t0 = w * CH
        pltpu.sync_copy(t_hbm.at[pl.ds(t0, CH)], tbuf)

        def add_rows(buf):
            @plsc.parallel_loop(0, CH, L, unroll=8)
            def _(k):
                plsc.addupdate(buf.at[pl.ds(k, L)], tbuf[pl.ds(k, L)])

        def outer(g, _):
            base = g * NBUF
            for b in range(NBUF):
                @pl.when(g > 0)
                def _():
                    pltpu.make_async_copy(
                        bufs[b],
                        o_hbm.at[pl.ds((base - NBUF + b) * P * D + t0, CH)],
                        out_sems[b],
                    ).wait()
                pltpu.async_copy(
                    x_hbm.at[pl.ds((base + b) * P * D + t0, CH)], bufs[b], in_sems[b]
                )
            for b in range(NBUF):
                pltpu.make_async_copy(
                    x_hbm.at[pl.ds((base + b) * P * D + t0, CH)], bufs[b], in_sems[b]
                ).wait()
                add_rows(bufs[b])
                pltpu.async_copy(
                    bufs[b], o_hbm.at[pl.ds((base + b) * P * D + t0, CH)], out_sems[b]
                )
            return 0

        lax.fori_loop(0, B // NBUF, outer, 0)
        for b in range(NBUF):
            pltpu.make_async_copy(
                bufs[b],
                o_hbm.at[pl.ds((B - NBUF + b) * P * D + t0, CH)],
                out_sems[b],
            ).wait()

    out = sc_add(encoded_patches.reshape(-1), pos_table.reshape(-1))
    return out.reshape(B, P, D)


# speedup vs baseline: 1.1929x; 1.1894x over previous
import functools
import jax
import jax.numpy as jnp
from jax import lax
from jax.experimental import pallas as pl
from jax.experimental.pallas import tpu as pltpu
from jax.experimental.pallas import tpu_sc as plsc

NBUF = 4
SC_BATCHES = 16


def _sc_part(encoded_patches, pos_table, S):
    B, P, D = encoded_patches.shape
    info = plsc.get_sparse_core_info()
    NC, NS, L = info.num_cores, info.num_subcores, info.num_lanes
    NW = NC * NS
    RP = P // NW
    CH = RP * D

    mesh = plsc.VectorSubcoreMesh(core_axis_name="c", subcore_axis_name="s")

    @functools.partial(
        pl.kernel,
        mesh=mesh,
        out_type=jax.ShapeDtypeStruct((S * P * D,), jnp.float32),
        scratch_types=(
            [pltpu.VMEM((CH,), jnp.float32)]
            + [pltpu.VMEM((CH,), jnp.float32)] * NBUF
            + [pltpu.SemaphoreType.DMA] * NBUF
            + [pltpu.SemaphoreType.DMA] * NBUF
        ),
    )
    def sc_add(x_hbm, t_hbm, o_hbm, tbuf, *rest):
        bufs = rest[:NBUF]
        in_sems = rest[NBUF:2 * NBUF]
        out_sems = rest[2 * NBUF:3 * NBUF]

        w = lax.axis_index("s") * NC + lax.axis_index("c")
        t0 = w * CH
        pltpu.sync_copy(t_hbm.at[pl.ds(t0, CH)], tbuf)

        def add_rows(buf):
            @plsc.parallel_loop(0, CH, L, unroll=8)
            def _(k):
                plsc.addupdate(buf.at[pl.ds(k, L)], tbuf[pl.ds(k, L)])

        def outer(g, _):
            base = g * NBUF
            for b in range(NBUF):
                @pl.when(g > 0)
                def _():
                    pltpu.make_async_copy(
                        bufs[b],
                        o_hbm.at[pl.ds((base - NBUF + b) * P * D + t0, CH)],
                        out_sems[b],
                    ).wait()
                pltpu.async_copy(
                    x_hbm.at[pl.ds((base + b) * P * D + t0, CH)], bufs[b], in_sems[b]
                )
            for b in range(NBUF):
                pltpu.make_async_copy(
                    x_hbm.at[pl.ds((base + b) * P * D + t0, CH)], bufs[b], in_sems[b]
                ).wait()
                add_rows(bufs[b])
                pltpu.async_copy(
                    bufs[b], o_hbm.at[pl.ds((base + b) * P * D + t0, CH)], out_sems[b]
                )
            return 0

        lax.fori_loop(0, S // NBUF, outer, 0)
        for b in range(NBUF):
            pltpu.make_async_copy(
                bufs[b],
                o_hbm.at[pl.ds((S - NBUF + b) * P * D + t0, CH)],
                out_sems[b],
            ).wait()

    out = sc_add(encoded_patches.reshape(-1), pos_table.reshape(-1))
    return out.reshape(S, P, D)


def _tc_add_kernel(x_ref, t_ref, o_ref):
    o_ref[...] = x_ref[...] + t_ref[...]


def _tc_part(encoded_patches, pos_table, S):
    B, P, D = encoded_patches.shape
    BB = 8
    off = S // BB
    return pl.pallas_call(
        _tc_add_kernel,
        grid=((B - S) // BB,),
        in_specs=[
            pl.BlockSpec((BB, P, D), lambda b: (b + off, 0, 0)),
            pl.BlockSpec((P, D), lambda b: (0, 0)),
        ],
        out_specs=pl.BlockSpec((BB, P, D), lambda b: (b, 0, 0)),
        out_shape=jax.ShapeDtypeStruct((B - S, P, D), encoded_patches.dtype),
    )(encoded_patches, pos_table)


def kernel(encoded_patches, pos_table):
    S = SC_BATCHES
    sc_out = _sc_part(encoded_patches, pos_table, S)
    tc_out = _tc_part(encoded_patches, pos_table, S)
    return jnp.concatenate([sc_out, tc_out], axis=0)


# speedup vs baseline: 5.0009x; 4.1921x over previous
import jax
import jax.numpy as jnp
from jax.experimental import pallas as pl
from jax.experimental.pallas import tpu as pltpu


def _add_kernel(x_ref, t_ref, o_ref):
    o_ref[...] = x_ref[...] + t_ref[...]


def kernel(encoded_patches, pos_table):
    B, P, D = encoded_patches.shape
    BB = 16
    PP = 192
    grid = (P // PP, B // BB)
    return pl.pallas_call(
        _add_kernel,
        grid=grid,
        in_specs=[
            pl.BlockSpec((BB, PP, D), lambda p, b: (b, p, 0)),
            pl.BlockSpec((PP, D), lambda p, b: (p, 0)),
        ],
        out_specs=pl.BlockSpec((BB, PP, D), lambda p, b: (b, p, 0)),
        out_shape=jax.ShapeDtypeStruct((B, P, D), encoded_patches.dtype),
    )(encoded_patches, pos_table)


# speedup vs baseline: 5.0706x; 1.0139x over previous
import jax
import jax.numpy as jnp
from jax.experimental import pallas as pl
from jax.experimental.pallas import tpu as pltpu


def _add_kernel(x_ref, t_ref, o_ref):
    o_ref[...] = x_ref[...] + t_ref[...]


def kernel(encoded_patches, pos_table):
    B, P, D = encoded_patches.shape
    BB = 8
    grid = (B // BB,)
    return pl.pallas_call(
        _add_kernel,
        grid=grid,
        in_specs=[
            pl.BlockSpec((BB, P, D), lambda b: (b, 0, 0)),
            pl.BlockSpec((P, D), lambda b: (0, 0)),
        ],
        out_specs=pl.BlockSpec((BB, P, D), lambda b: (b, 0, 0)),
        out_shape=jax.ShapeDtypeStruct((B, P, D), encoded_patches.dtype),
    )(encoded_patches, pos_table)
